# baseline (speedup 1.0000x reference)
import math

import jax
import jax.numpy as jnp
from jax.experimental import pallas as pl

EMBED_DIM = 256
NUM_POS_FEATS = EMBED_DIM // 2
IMAGE_SIZE = (1024, 1024)
NUM_MASKS = 4


def _encoder_body(points_ref, feats_ref, pe_ref, corner_ref, point_ref,
                  attr_ref, mask_ref, out_ref):
    pts = points_ref[...].reshape(-1, 4)
    feats = feats_ref[...].reshape(-1, EMBED_DIM)
    g0 = pe_ref[0]
    g1 = pe_ref[1]
    base = point_ref[0, 0] + attr_ref[1]

    two_pi = 2.0 * math.pi
    sx = two_pi * (2.0 / IMAGE_SIZE[1])
    sy = two_pi * (2.0 / IMAGE_SIZE[0])

    q = pts.shape[0]
    for k in range(2):
        x = pts[:, 2 * k] * sx - two_pi
        y = pts[:, 2 * k + 1] * sy - two_pi
        arg = x[:, None] * g0[None, :] + y[:, None] * g1[None, :]
        pe = jnp.concatenate([jnp.sin(arg), jnp.cos(arg)], axis=-1)
        v = pe + (base + corner_ref[0, k])[None, :] + feats
        out_ref[0, :, k, :] = v[:q // 2]
        out_ref[1, :, k, :] = v[q // 2:]
    bc = jnp.broadcast_to(mask_ref[0][None], (q // 2, NUM_MASKS + 1, EMBED_DIM))
    out_ref[0, :, 2:, :] = bc
    out_ref[1, :, 2:, :] = bc


def kernel(points, feats_centers, pe_gaussian, corner_emb, point_emb, attr_W, mask_emb):
    B, Q, _ = points.shape
    C = EMBED_DIM
    S = 2 + NUM_MASKS + 1
    out = pl.pallas_call(
        _encoder_body,
        grid=(B // 2,),
        in_specs=[
            pl.BlockSpec((2, Q, 4), lambda b: (b, 0, 0)),
            pl.BlockSpec((2, Q, C), lambda b: (b, 0, 0)),
            pl.BlockSpec((2, NUM_POS_FEATS), lambda b: (0, 0)),
            pl.BlockSpec((1, 2, C), lambda b: (0, 0, 0)),
            pl.BlockSpec((1, 1, C), lambda b: (0, 0, 0)),
            pl.BlockSpec((2, C), lambda b: (0, 0)),
            pl.BlockSpec((1, NUM_MASKS + 1, C), lambda b: (0, 0, 0)),
        ],
        out_specs=pl.BlockSpec((2, Q, S, C), lambda b: (b, 0, 0, 0)),
        out_shape=jax.ShapeDtypeStruct((B, Q, S, C), jnp.float32),
    )(points, feats_centers, pe_gaussian, corner_emb, point_emb, attr_W, mask_emb)
    return (out, out)
